# baseline (speedup 1.0000x reference)
import jax
import jax.numpy as jnp
from jax import lax
from jax.experimental import pallas as pl
from jax.experimental.pallas import tpu as pltpu
from jax.experimental.pallas import tpu_sc as plsc

_NC, _NS, _L = 2, 16, 16
_B = 256
_N1, _N2 = 2000, 500
_N1P, _N2P = 2048, 512
_H = 2
_CH = 128
_SUP = 4


def _make_sc_segment_sum(n_rows, lanes, e_pad, sup, gw):
  per_core = e_pad // _NC
  n_chunks = per_core // _CH
  q = lanes // _L
  pr = gw // lanes
  n_sup = n_chunks // sup
  sup_groups = sup * _CH // _L

  mesh = plsc.VectorSubcoreMesh(core_axis_name="c", subcore_axis_name="s",
                                num_cores=_NC, num_subcores=_NS)

  def body(table, src, dst, out_sum, out_cnt,
           src_v, dst_v, rows_v, acc, cacc,
           semg0, semg1, semi0, semi1):
    c = lax.axis_index("c")
    s = lax.axis_index("s")
    zrow = jnp.zeros((_L,), jnp.float32)
    ones = jnp.ones((_L,), jnp.float32)
    iota = lax.iota(jnp.int32, _L)
    kvecs = [jnp.full((_L,), k, jnp.int32) for k in range(_L)]
    semg = [semg0, semg1]
    semi = [semi0, semi1]

    def zero(i, carry):
      for u in range(q):
        acc[i, pl.ds(u * _L, _L)] = zrow
      cacc[i] = zrow
      return carry

    lax.fori_loop(0, n_rows, zero, 0)

    cbase = c * n_chunks
    tbl = table.at[s // pr]
    coff = (s % pr) * lanes

    def issue_idx(slot, g):
      cb = cbase + g * sup
      pltpu.async_copy(src.at[pl.ds(cb, sup)], src_v.at[slot], semi[slot])
      pltpu.async_copy(dst.at[pl.ds(cb, sup)], dst_v.at[slot], semi[slot])

    def drain_idx(slot):
      pltpu.make_async_copy(src.at[pl.ds(0, sup)], src_v.at[slot],
                            semi[slot]).wait()
      pltpu.make_async_copy(dst.at[pl.ds(0, sup)], dst_v.at[slot],
                            semi[slot]).wait()

    def fire_gathers(slot):
      for i in range(sup):
        pltpu.async_copy(tbl.at[src_v.at[slot, i]], rows_v.at[slot, i],
                         semg[slot])

    def drain_gathers(slot):
      for i in range(sup):
        pltpu.make_async_copy(tbl.at[src_v.at[slot, i]], rows_v.at[slot, i],
                              semg[slot]).wait()

    def compute(slot, g):
      def group(gg, carry2):
        i = gg >> 3
        e0 = (gg & 7) * _L
        dvec = dst_v[slot, i, pl.ds(e0, _L)]
        dbks = []
        vals = []
        for k in range(_L):
          dbks.append(lax.gather(
              dvec, kvecs[k][:, None],
              lax.GatherDimensionNumbers(offset_dims=(),
                                         collapsed_slice_dims=(0,),
                                         start_index_map=(0,)),
              slice_sizes=(1,),
              mode=lax.GatherScatterMode.PROMISE_IN_BOUNDS))
          vals.append([rows_v[slot, i, e0 + k, pl.ds(coff + u * _L, _L)]
                       for u in range(q)])
        for k in range(_L):
          for u in range(q):
            plsc.addupdate_scatter(acc, [dbks[k], iota + u * _L], vals[k][u])

        @pl.when((gg & 15) == s)
        def _():
          plsc.addupdate_scatter(cacc, [dvec, dvec & 15], ones)

        return carry2

      lax.fori_loop(0, sup_groups, group, 0)

    issue_idx(0, 0)
    drain_idx(0)
    fire_gathers(0)
    issue_idx(1, 1)

    def step(g, carry):
      for slot in range(2):
        @pl.when((g & 1) == slot)
        def _():
          other = 1 - slot
          drain_gathers(slot)

          @pl.when(g + 1 < n_sup)
          def _():
            drain_idx(other)
            fire_gathers(other)

          compute(slot, g)

          @pl.when(g + 2 < n_sup)
          def _():
            issue_idx(slot, g + 2)

      return carry

    lax.fori_loop(0, n_sup, step, 0)
    pltpu.sync_copy(acc, out_sum.at[c, s])
    pltpu.sync_copy(cacc, out_cnt.at[c, s])

  return pl.kernel(
      body,
      out_type=(jax.ShapeDtypeStruct((_NC, _NS, n_rows, lanes), jnp.float32),
                jax.ShapeDtypeStruct((_NC, _NS, n_rows, 16), jnp.float32)),
      mesh=mesh,
      compiler_params=pltpu.CompilerParams(use_tc_tiling_on_sc=False,
                                           needs_layout_passes=False),
      scratch_types=[
          pltpu.VMEM((2, sup, _CH), jnp.int32),
          pltpu.VMEM((2, sup, _CH), jnp.int32),
          pltpu.VMEM((2, sup, _CH, gw), jnp.float32),
          pltpu.VMEM((n_rows, lanes), jnp.float32),
          pltpu.VMEM((n_rows, 16), jnp.float32),
          pltpu.SemaphoreType.DMA,
          pltpu.SemaphoreType.DMA,
          pltpu.SemaphoreType.DMA,
          pltpu.SemaphoreType.DMA,
      ],
  )


def _unswizzle_cnt(c_ref):
  cnt16 = jnp.sum(jnp.sum(c_ref[...], axis=0), axis=0)
  blk = cnt16.shape[0]
  rid = lax.broadcasted_iota(jnp.int32, (blk, 16), 0)
  cid = lax.broadcasted_iota(jnp.int32, (blk, 16), 1)
  mask = (cid == (rid % 16)).astype(jnp.float32)
  return jnp.maximum(jnp.sum(cnt16 * mask, axis=1, keepdims=True), 1.0)


def _layer1_body(s_ref, c_ref, w_ref, y_ref):
  cnt = _unswizzle_cnt(c_ref)
  gs = [(s_ref[0, s] + s_ref[1, s]) / cnt for s in range(_NS)]
  gmn = gs[0].min(axis=1, keepdims=True)
  gmx = gs[0].max(axis=1, keepdims=True)
  for s in range(1, _NS):
    gmn = jnp.minimum(gmn, gs[s].min(axis=1, keepdims=True))
    gmx = jnp.maximum(gmx, gs[s].max(axis=1, keepdims=True))
  for h in range(_H):
    w = w_ref[0, h]
    mn = jnp.minimum(w * gmn, w * gmx)
    mx = jnp.maximum(w * gmn, w * gmx)
    inv = 1.0 / (mx - mn + 1e-8)
    for s in range(_NS):
      col = (s & 3) * 2 * _L + h * _L
      y_ref[s >> 2, :, col:col + _L] = jnp.maximum(
          (gs[s] * w - mn) * inv, 0.0)


def _layer2_body(s_ref, c_ref, w_ref, a0_ref, a1_ref, b_ref, o_ref):
  cnt = _unswizzle_cnt(c_ref)
  g0 = jnp.concatenate(
      [s_ref[0, s, :, 0:_L] + s_ref[1, s, :, 0:_L] for s in range(_NS)],
      axis=1) / cnt
  g1 = jnp.concatenate(
      [s_ref[0, s, :, _L:2 * _L] + s_ref[1, s, :, _L:2 * _L]
       for s in range(_NS)], axis=1) / cnt
  ah0 = g0 * w_ref[0, 0] + g1 * w_ref[1, 0]
  ah1 = g0 * w_ref[0, 1] + g1 * w_ref[1, 1]

  def _mm(v):
    mn = jnp.min(v, axis=1, keepdims=True)
    mx = jnp.max(v, axis=1, keepdims=True)
    return jnp.maximum((v - mn) / (mx - mn + 1e-8), 0.0)

  x0 = _mm(ah0)
  x1 = _mm(ah1)
  acc = lax.dot_general(x0, a0_ref[...], (((0,), (0,)), ((), ())),
                        preferred_element_type=jnp.float32,
                        precision=lax.Precision.HIGHEST)
  acc = acc + lax.dot_general(x1, a1_ref[...], (((0,), (0,)), ((), ())),
                              preferred_element_type=jnp.float32,
                              precision=lax.Precision.HIGHEST)
  o_ref[...] = acc + b_ref[...]


def _pad_edges(edge_index, e_pad, n_src, scrap_dst):
  e = edge_index.shape[1]
  pad = e_pad - e
  pad_src = (jnp.arange(pad, dtype=jnp.int32) % n_src)
  src = jnp.concatenate([edge_index[0], pad_src])
  dst = jnp.concatenate(
      [edge_index[1], jnp.full((pad,), scrap_dst, jnp.int32)])
  return src, dst


def kernel(data, edge_index_1, edge_index_2, W1, W2, W_out, b_out):
  n0 = data.shape[1]
  e1, e2 = edge_index_1.shape[1], edge_index_2.shape[1]
  step = _NC * _SUP * _CH
  e1p = -(-e1 // step) * step
  e2p = -(-e2 // step) * step

  data_cs = jnp.transpose(data.reshape(_NS, _L, n0), (0, 2, 1))
  src1, dst1 = _pad_edges(edge_index_1, e1p, n0, _N1P - 1)
  src2, dst2 = _pad_edges(edge_index_2, e2p, _N1, _N2P - 1)

  seg1 = _make_sc_segment_sum(_N1P, _L, e1p, _SUP, _L)
  sums1, cnts1 = seg1(data_cs, src1.reshape(-1, _CH), dst1.reshape(-1, _CH))

  blk = 256
  y1 = pl.pallas_call(
      _layer1_body,
      grid=(_N1P // blk,),
      in_specs=[pl.BlockSpec((_NC, _NS, blk, _L), lambda i: (0, 0, i, 0)),
                pl.BlockSpec((_NC, _NS, blk, 16), lambda i: (0, 0, i, 0)),
                pl.BlockSpec((1, _H), lambda i: (0, 0))],
      out_specs=pl.BlockSpec((4, blk, 128), lambda i: (0, i, 0)),
      out_shape=jax.ShapeDtypeStruct((4, _N1P, 128), jnp.float32),
  )(sums1, cnts1, W1)

  seg2 = _make_sc_segment_sum(_N2P, _H * _L, e2p, 2, 128)
  sums2, cnts2 = seg2(y1, src2.reshape(-1, _CH), dst2.reshape(-1, _CH))

  c = W_out.shape[1]
  a0 = jnp.zeros((_N2P, c), jnp.float32).at[:_N2].set(W_out[0::2])
  a1 = jnp.zeros((_N2P, c), jnp.float32).at[:_N2].set(W_out[1::2])
  out = pl.pallas_call(
      _layer2_body,
      out_shape=jax.ShapeDtypeStruct((_B, c), jnp.float32),
  )(sums2, cnts2, W2, a0, a1, b_out.reshape(1, c))
  return out


# speedup vs baseline: 1.0430x; 1.0430x over previous
import jax
import jax.numpy as jnp
from jax import lax
from jax.experimental import pallas as pl
from jax.experimental.pallas import tpu as pltpu
from jax.experimental.pallas import tpu_sc as plsc

_NC, _NS, _L = 2, 16, 16
_B = 256
_N1, _N2 = 2000, 500
_N1P, _N2P = 2048, 512
_H = 2
_CH = 128
_SUP = 4


def _make_sc_segment_sum(n_rows, lanes, e_pad):
  per_core = e_pad // _NC
  n_chunks = per_core // _CH
  q = lanes // _L
  n_sup = n_chunks // _SUP
  sup_groups = _SUP * _CH // _L

  mesh = plsc.VectorSubcoreMesh(core_axis_name="c", subcore_axis_name="s",
                                num_cores=_NC, num_subcores=_NS)

  def body(table, src, dst, out_sum, out_cnt,
           src_v, dst_v, rows_v, acc, cacc,
           semg0, semg1, semi0, semi1):
    c = lax.axis_index("c")
    s = lax.axis_index("s")
    zrow = jnp.zeros((_L,), jnp.float32)
    ones = jnp.ones((_L,), jnp.float32)
    iota = lax.iota(jnp.int32, _L)
    kvecs = [jnp.full((_L,), k, jnp.int32) for k in range(_L)]
    semg = [semg0, semg1]
    semi = [semi0, semi1]

    def zero(i, carry):
      for u in range(q):
        acc[i, pl.ds(u * _L, _L)] = zrow
      cacc[i] = zrow
      return carry

    lax.fori_loop(0, n_rows, zero, 0)

    cbase = c * n_chunks
    tbl = table.at[s]

    def issue_idx(slot, g):
      cb = cbase + g * _SUP
      pltpu.async_copy(src.at[pl.ds(cb, _SUP)], src_v.at[slot], semi[slot])
      pltpu.async_copy(dst.at[pl.ds(cb, _SUP)], dst_v.at[slot], semi[slot])

    def drain_idx(slot):
      pltpu.make_async_copy(src.at[pl.ds(0, _SUP)], src_v.at[slot],
                            semi[slot]).wait()
      pltpu.make_async_copy(dst.at[pl.ds(0, _SUP)], dst_v.at[slot],
                            semi[slot]).wait()

    def fire_gathers(slot):
      for i in range(_SUP):
        pltpu.async_copy(tbl.at[src_v.at[slot, i]], rows_v.at[slot, i],
                         semg[slot])

    def drain_gathers(slot):
      for i in range(_SUP):
        pltpu.make_async_copy(tbl.at[src_v.at[slot, i]], rows_v.at[slot, i],
                              semg[slot]).wait()

    def compute(slot, g):
      def group(gg, carry2):
        i = gg >> 3
        e0 = (gg & 7) * _L
        dvec = dst_v[slot, i, pl.ds(e0, _L)]
        dbks = []
        vals = []
        for k in range(_L):
          dbks.append(lax.gather(
              dvec, kvecs[k][:, None],
              lax.GatherDimensionNumbers(offset_dims=(),
                                         collapsed_slice_dims=(0,),
                                         start_index_map=(0,)),
              slice_sizes=(1,),
              mode=lax.GatherScatterMode.PROMISE_IN_BOUNDS))
          vals.append([rows_v[slot, i, e0 + k, pl.ds(u * _L, _L)]
                       for u in range(q)])
        for k in range(_L):
          for u in range(q):
            plsc.addupdate_scatter(acc, [dbks[k], iota + u * _L], vals[k][u])

        @pl.when((gg & 15) == s)
        def _():
          plsc.addupdate_scatter(cacc, [dvec, dvec & 15], ones)

        return carry2

      lax.fori_loop(0, sup_groups, group, 0)

    issue_idx(0, 0)
    drain_idx(0)
    fire_gathers(0)
    issue_idx(1, 1)

    def step(g, carry):
      for slot in range(2):
        @pl.when((g & 1) == slot)
        def _():
          other = 1 - slot
          drain_gathers(slot)

          @pl.when(g + 1 < n_sup)
          def _():
            drain_idx(other)
            fire_gathers(other)

          compute(slot, g)

          @pl.when(g + 2 < n_sup)
          def _():
            issue_idx(slot, g + 2)

      return carry

    lax.fori_loop(0, n_sup, step, 0)
    pltpu.sync_copy(acc, out_sum.at[c, s])
    pltpu.sync_copy(cacc, out_cnt.at[c, s])

  return pl.kernel(
      body,
      out_type=(jax.ShapeDtypeStruct((_NC, _NS, n_rows, lanes), jnp.float32),
                jax.ShapeDtypeStruct((_NC, _NS, n_rows, 16), jnp.float32)),
      mesh=mesh,
      compiler_params=pltpu.CompilerParams(use_tc_tiling_on_sc=False,
                                           needs_layout_passes=False),
      scratch_types=[
          pltpu.VMEM((2, _SUP, _CH), jnp.int32),
          pltpu.VMEM((2, _SUP, _CH), jnp.int32),
          pltpu.VMEM((2, _SUP, _CH, lanes), jnp.float32),
          pltpu.VMEM((n_rows, lanes), jnp.float32),
          pltpu.VMEM((n_rows, 16), jnp.float32),
          pltpu.SemaphoreType.DMA,
          pltpu.SemaphoreType.DMA,
          pltpu.SemaphoreType.DMA,
          pltpu.SemaphoreType.DMA,
      ],
  )


def _unswizzle_cnt(c_ref):
  cnt16 = jnp.sum(jnp.sum(c_ref[...], axis=0), axis=0)
  blk = cnt16.shape[0]
  rid = lax.broadcasted_iota(jnp.int32, (blk, 16), 0)
  cid = lax.broadcasted_iota(jnp.int32, (blk, 16), 1)
  mask = (cid == (rid % 16)).astype(jnp.float32)
  return jnp.maximum(jnp.sum(cnt16 * mask, axis=1, keepdims=True), 1.0)


def _layer1_body(s_ref, c_ref, w_ref, y_ref):
  cnt = _unswizzle_cnt(c_ref)
  gs = [(s_ref[0, s] + s_ref[1, s]) / cnt for s in range(_NS)]
  gmn = gs[0].min(axis=1, keepdims=True)
  gmx = gs[0].max(axis=1, keepdims=True)
  for s in range(1, _NS):
    gmn = jnp.minimum(gmn, gs[s].min(axis=1, keepdims=True))
    gmx = jnp.maximum(gmx, gs[s].max(axis=1, keepdims=True))
  for h in range(_H):
    w = w_ref[0, h]
    mn = jnp.minimum(w * gmn, w * gmx)
    mx = jnp.maximum(w * gmn, w * gmx)
    inv = 1.0 / (mx - mn + 1e-8)
    for s in range(_NS):
      y_ref[s, :, h * _L:(h + 1) * _L] = jnp.maximum(
          (gs[s] * w - mn) * inv, 0.0)


def _layer2_body(s_ref, c_ref, w_ref, a0_ref, a1_ref, b_ref, o_ref):
  cnt = _unswizzle_cnt(c_ref)
  g0 = jnp.concatenate(
      [s_ref[0, s, :, 0:_L] + s_ref[1, s, :, 0:_L] for s in range(_NS)],
      axis=1) / cnt
  g1 = jnp.concatenate(
      [s_ref[0, s, :, _L:2 * _L] + s_ref[1, s, :, _L:2 * _L]
       for s in range(_NS)], axis=1) / cnt
  ah0 = g0 * w_ref[0, 0] + g1 * w_ref[1, 0]
  ah1 = g0 * w_ref[0, 1] + g1 * w_ref[1, 1]

  def _mm(v):
    mn = jnp.min(v, axis=1, keepdims=True)
    mx = jnp.max(v, axis=1, keepdims=True)
    return jnp.maximum((v - mn) / (mx - mn + 1e-8), 0.0)

  x0 = _mm(ah0)
  x1 = _mm(ah1)
  acc = lax.dot_general(x0, a0_ref[...], (((0,), (0,)), ((), ())),
                        preferred_element_type=jnp.float32,
                        precision=lax.Precision.HIGHEST)
  acc = acc + lax.dot_general(x1, a1_ref[...], (((0,), (0,)), ((), ())),
                              preferred_element_type=jnp.float32,
                              precision=lax.Precision.HIGHEST)
  o_ref[...] = acc + b_ref[...]


def _pad_edges(edge_index, e_pad, n_src, scrap_dst):
  e = edge_index.shape[1]
  pad = e_pad - e
  pad_src = (jnp.arange(pad, dtype=jnp.int32) % n_src)
  src = jnp.concatenate([edge_index[0], pad_src])
  dst = jnp.concatenate(
      [edge_index[1], jnp.full((pad,), scrap_dst, jnp.int32)])
  return src, dst


def kernel(data, edge_index_1, edge_index_2, W1, W2, W_out, b_out):
  n0 = data.shape[1]
  e1, e2 = edge_index_1.shape[1], edge_index_2.shape[1]
  step = _NC * _SUP * _CH
  e1p = -(-e1 // step) * step
  e2p = -(-e2 // step) * step

  data_cs = jnp.transpose(data.reshape(_NS, _L, n0), (0, 2, 1))
  src1, dst1 = _pad_edges(edge_index_1, e1p, n0, _N1P - 1)
  src2, dst2 = _pad_edges(edge_index_2, e2p, _N1, _N2P - 1)

  seg1 = _make_sc_segment_sum(_N1P, _L, e1p)
  sums1, cnts1 = seg1(data_cs, src1.reshape(-1, _CH), dst1.reshape(-1, _CH))

  blk = 256
  y1 = pl.pallas_call(
      _layer1_body,
      grid=(_N1P // blk,),
      in_specs=[pl.BlockSpec((_NC, _NS, blk, _L), lambda i: (0, 0, i, 0)),
                pl.BlockSpec((_NC, _NS, blk, 16), lambda i: (0, 0, i, 0)),
                pl.BlockSpec((1, _H), lambda i: (0, 0))],
      out_specs=pl.BlockSpec((_NS, blk, _H * _L), lambda i: (0, i, 0)),
      out_shape=jax.ShapeDtypeStruct((_NS, _N1P, _H * _L), jnp.float32),
  )(sums1, cnts1, W1)

  seg2 = _make_sc_segment_sum(_N2P, _H * _L, e2p)
  sums2, cnts2 = seg2(y1, src2.reshape(-1, _CH), dst2.reshape(-1, _CH))

  c = W_out.shape[1]
  a0 = jnp.zeros((_N2P, c), jnp.float32).at[:_N2].set(W_out[0::2])
  a1 = jnp.zeros((_N2P, c), jnp.float32).at[:_N2].set(W_out[1::2])
  out = pl.pallas_call(
      _layer2_body,
      out_shape=jax.ShapeDtypeStruct((_B, c), jnp.float32),
  )(sums2, cnts2, W2, a0, a1, b_out.reshape(1, c))
  return out
